# baseline (speedup 1.0000x reference)
import functools
import jax
import jax.numpy as jnp
from jax.experimental import pallas as pl

H = 16
DH = 64
BUCKET = 64
NHASH = 4
RADIUS = 4
NB2 = 16
SCALE = 8.0

BM = 256


def _dense_body(x_ref, w_ref, b_ref, o_ref, *, relu, bf16):
    x = x_ref[...]
    w = w_ref[...]
    if bf16:
        x = x.astype(jnp.bfloat16)
        w = w.astype(jnp.bfloat16)
    acc = jnp.dot(x, w, preferred_element_type=jnp.float32)
    acc = acc + b_ref[0][None, :]
    if relu:
        acc = jnp.maximum(acc, 0.0)
    o_ref[...] = acc


def _dense(x, w, b, relu=False, bn=None, bf16=False):
    M, K = x.shape
    K2, N = w.shape
    assert K == K2
    if bn is None:
        bn = N
    grid = (M // BM, N // bn)
    return pl.pallas_call(
        functools.partial(_dense_body, relu=relu, bf16=bf16),
        grid=grid,
        in_specs=[
            pl.BlockSpec((BM, K), lambda i, j: (i, 0)),
            pl.BlockSpec((K, bn), lambda i, j: (0, j)),
            pl.BlockSpec((1, bn), lambda i, j: (0, j)),
        ],
        out_specs=pl.BlockSpec((BM, bn), lambda i, j: (i, j)),
        out_shape=jax.ShapeDtypeStruct((M, N), jnp.float32),
    )(x, w, b.reshape(1, N))


def _ln_dense_body(x_ref, g_ref, bt_ref, w_ref, b_ref, o_ref):
    xb = x_ref[...]
    mu = jnp.mean(xb, axis=-1, keepdims=True)
    xc = xb - mu
    var = jnp.mean(xc * xc, axis=-1, keepdims=True)
    xn = xc * jax.lax.rsqrt(var + 1e-5) * g_ref[0][None, :] + bt_ref[0][None, :]
    acc = jnp.dot(xn.astype(jnp.bfloat16), w_ref[...].astype(jnp.bfloat16),
                  preferred_element_type=jnp.float32)
    acc = acc + b_ref[0][None, :]
    o_ref[...] = jnp.maximum(acc, 0.0)


def _ln_dense_relu(x, ln_g, ln_b, w, b):
    M, K = x.shape
    _, N = w.shape
    grid = (M // BM,)
    return pl.pallas_call(
        _ln_dense_body,
        grid=grid,
        in_specs=[
            pl.BlockSpec((BM, K), lambda i: (i, 0)),
            pl.BlockSpec((1, K), lambda i: (0, 0)),
            pl.BlockSpec((1, K), lambda i: (0, 0)),
            pl.BlockSpec((K, N), lambda i: (0, 0)),
            pl.BlockSpec((1, N), lambda i: (0, 0)),
        ],
        out_specs=pl.BlockSpec((BM, N), lambda i: (i, 0)),
        out_shape=jax.ShapeDtypeStruct((M, N), jnp.float32),
    )(x, ln_g.reshape(1, K), ln_b.reshape(1, K), w, b.reshape(1, N))


def _bucket_body(qk_ref, r_ref, o_ref):
    qh = qk_ref[0]
    outs = []
    for n in range(NHASH):
        rot = jnp.dot(qh, r_ref[n], preferred_element_type=jnp.float32)
        both = jnp.concatenate([rot, -rot], axis=-1)
        outs.append(jnp.argmax(both, axis=-1).astype(jnp.int32))
    o_ref[0] = jnp.stack(outs, axis=0)


def _buckets(qkh, R):
    Hh, S, _ = qkh.shape
    return pl.pallas_call(
        _bucket_body,
        grid=(Hh,),
        in_specs=[
            pl.BlockSpec((1, S, DH), lambda h: (h, 0, 0)),
            pl.BlockSpec((NHASH, DH, NB2), lambda h: (0, 0, 0)),
        ],
        out_specs=pl.BlockSpec((1, NHASH, S), lambda h: (h, 0, 0)),
        out_shape=jax.ShapeDtypeStruct((Hh, NHASH, S), jnp.int32),
    )(qkh, R)


def _lsh_body(q_ref, v_ref, o_ref, *, S):
    q = q_ref[0]
    v = v_ref[0]
    n = jnp.sqrt(jnp.sum(q * q, axis=-1, keepdims=True))
    k = q / (n + 1e-6)
    nch = S // BUCKET
    for c in range(nch):
        p = (c + nch - 1) % nch
        qc = q[c * BUCKET:(c + 1) * BUCKET]
        k2 = jnp.concatenate(
            [k[c * BUCKET:(c + 1) * BUCKET], k[p * BUCKET:(p + 1) * BUCKET]],
            axis=0)
        v2 = jnp.concatenate(
            [v[c * BUCKET:(c + 1) * BUCKET], v[p * BUCKET:(p + 1) * BUCKET]],
            axis=0)
        sc = jax.lax.dot_general(
            qc, k2, (((1,), (1,)), ((), ())),
            preferred_element_type=jnp.float32) / SCALE
        m = jnp.max(sc, axis=-1, keepdims=True)
        e = jnp.exp(sc - m)
        a = e / jnp.sum(e, axis=-1, keepdims=True)
        o_ref[0, c * BUCKET:(c + 1) * BUCKET, :] = jnp.dot(
            a, v2, preferred_element_type=jnp.float32)


def _lsh_attn_sorted(sq, sv):
    G, S, _ = sq.shape
    return pl.pallas_call(
        functools.partial(_lsh_body, S=S),
        grid=(G,),
        in_specs=[
            pl.BlockSpec((1, S, DH), lambda g: (g, 0, 0)),
            pl.BlockSpec((1, S, DH), lambda g: (g, 0, 0)),
        ],
        out_specs=pl.BlockSpec((1, S, DH), lambda g: (g, 0, 0)),
        out_shape=jax.ShapeDtypeStruct((G, S, DH), jnp.float32),
    )(sq, sv)


def _roll_rows(a, t):
    t = t % a.shape[0]
    if t == 0:
        return a
    return jnp.concatenate([a[-t:], a[:-t]], axis=0)


def _local_body(q_ref, v_ref, o_ref, *, S):
    q = q_ref[0]
    v = v_ref[0]
    n = jnp.sqrt(jnp.sum(q * q, axis=-1, keepdims=True))
    k = q / (n + 1e-6)
    pos = jax.lax.broadcasted_iota(jnp.int32, (S, 1), 0)
    shifts = list(range(-RADIUS, RADIUS + 1))
    scs = []
    for s in shifts:
        kw = _roll_rows(k, -s)
        sc = jnp.sum(q * kw, axis=-1, keepdims=True) / SCALE
        valid = (pos + s >= 0) & (pos + s < S)
        scs.append(jnp.where(valid, sc, -1e9))
    m = scs[0]
    for sc in scs[1:]:
        m = jnp.maximum(m, sc)
    es = [jnp.exp(sc - m) for sc in scs]
    denom = es[0]
    for e in es[1:]:
        denom = denom + e
    acc = jnp.zeros_like(v)
    for s, e in zip(shifts, es):
        acc = acc + (e / denom) * _roll_rows(v, -s)
    o_ref[0] = acc


def _local_attn(qkh, vh):
    Hh, S, _ = qkh.shape
    return pl.pallas_call(
        functools.partial(_local_body, S=S),
        grid=(Hh,),
        in_specs=[
            pl.BlockSpec((1, S, DH), lambda h: (h, 0, 0)),
            pl.BlockSpec((1, S, DH), lambda h: (h, 0, 0)),
        ],
        out_specs=pl.BlockSpec((1, S, DH), lambda h: (h, 0, 0)),
        out_shape=jax.ShapeDtypeStruct((Hh, S, DH), jnp.float32),
    )(qkh, vh)


def _heads(x2d, W, b0, S, bf16=False):
    y = _dense(x2d, W, b0, bf16=bf16)
    return y.reshape(S, H, DH).transpose(1, 0, 2)


def _attn_layer(x2d, lp, S, zeros_d):
    qkh = _heads(x2d, lp['Wqk'], zeros_d, S)
    vh = _heads(x2d, lp['Wv'], zeros_d, S, bf16=True)

    bks = _buckets(qkh, lp['R'])
    bks = bks.transpose(1, 0, 2)
    ticker = jnp.arange(S, dtype=jnp.int32)
    perm = jnp.argsort(bks * S + ticker[None, None, :], axis=-1)
    inv = jnp.argsort(perm, axis=-1)

    sq = jnp.take_along_axis(
        jnp.broadcast_to(qkh[None], (NHASH, H, S, DH)), perm[..., None], axis=2)
    sv = jnp.take_along_axis(
        jnp.broadcast_to(vh[None], (NHASH, H, S, DH)), perm[..., None], axis=2)

    so = _lsh_attn_sorted(sq.reshape(NHASH * H, S, DH),
                          sv.reshape(NHASH * H, S, DH))
    so = so.reshape(NHASH, H, S, DH)
    o = jnp.take_along_axis(so, inv[..., None], axis=2)
    lsh = jnp.mean(o, axis=0)

    loc = _local_attn(qkh, vh)

    g = jax.nn.sigmoid(lp['router'][:, :S])
    comb = g[:, :, None] * lsh + (1.0 - g)[:, :, None] * loc
    o2d = comb.transpose(1, 0, 2).reshape(S, H * DH)
    out = _dense(o2d, lp['Wo'], zeros_d, bf16=True)
    reg = jnp.mean(g * (1.0 - g))
    return out, reg


def _run(src, tgt, params):
    S = src.shape[1]
    D = params['emb'].shape[1]
    x = params['emb'][src[0]] + params['pos'][0, :S, :]
    zeros_d = jnp.zeros((D,), jnp.float32)

    x1 = x
    x2 = jnp.zeros_like(x)
    total_reg = jnp.zeros((), jnp.float32)
    for lp in params['layers']:
        a, reg = _attn_layer(x2, lp, S, zeros_d)
        y1 = x1 + jax.nn.sigmoid(lp['gf'])[None, :] * a
        h = _ln_dense_relu(y1, lp['ln_g'], lp['ln_b'], lp['W1'], lp['b1'])
        ffn = _dense(h, lp['W2'], lp['b2'], bf16=True)
        y2 = x2 + jax.nn.sigmoid(lp['gg'])[None, :] * ffn
        x1, x2 = y1, y2
        total_reg = total_reg + reg

    out = (x1 + x2) / 2.0
    out = out[: tgt.shape[1], :]
    logits = _dense(out, params['Wout'], params['bout'], bn=1280, bf16=True)
    return logits[None], total_reg


@jax.jit
def kernel(src, tgt, params):
    return _run(src, tgt, params)


# speedup vs baseline: 1.2448x; 1.2448x over previous
import functools
import jax
import jax.numpy as jnp
from jax.experimental import pallas as pl

H = 16
DH = 64
BUCKET = 64
NHASH = 4
RADIUS = 4
NB2 = 16
SCALE = 8.0

BM = 256


def _dense_body(x_ref, w_ref, b_ref, o_ref, *, relu, bf16):
    x = x_ref[...]
    w = w_ref[...]
    if bf16:
        x = x.astype(jnp.bfloat16)
        w = w.astype(jnp.bfloat16)
    acc = jnp.dot(x, w, preferred_element_type=jnp.float32)
    acc = acc + b_ref[0][None, :]
    if relu:
        acc = jnp.maximum(acc, 0.0)
    o_ref[...] = acc


def _dense(x, w, b, relu=False, bn=None, bf16=False):
    M, K = x.shape
    K2, N = w.shape
    assert K == K2
    if bn is None:
        bn = N
    grid = (M // BM, N // bn)
    return pl.pallas_call(
        functools.partial(_dense_body, relu=relu, bf16=bf16),
        grid=grid,
        in_specs=[
            pl.BlockSpec((BM, K), lambda i, j: (i, 0)),
            pl.BlockSpec((K, bn), lambda i, j: (0, j)),
            pl.BlockSpec((1, bn), lambda i, j: (0, j)),
        ],
        out_specs=pl.BlockSpec((BM, bn), lambda i, j: (i, j)),
        out_shape=jax.ShapeDtypeStruct((M, N), jnp.float32),
    )(x, w, b.reshape(1, N))


def _ln_dense_body(x_ref, g_ref, bt_ref, w_ref, b_ref, o_ref):
    xb = x_ref[...]
    mu = jnp.mean(xb, axis=-1, keepdims=True)
    xc = xb - mu
    var = jnp.mean(xc * xc, axis=-1, keepdims=True)
    xn = xc * jax.lax.rsqrt(var + 1e-5) * g_ref[0][None, :] + bt_ref[0][None, :]
    acc = jnp.dot(xn.astype(jnp.bfloat16), w_ref[...].astype(jnp.bfloat16),
                  preferred_element_type=jnp.float32)
    acc = acc + b_ref[0][None, :]
    o_ref[...] = jnp.maximum(acc, 0.0)


def _ln_dense_relu(x, ln_g, ln_b, w, b):
    M, K = x.shape
    _, N = w.shape
    grid = (M // BM,)
    return pl.pallas_call(
        _ln_dense_body,
        grid=grid,
        in_specs=[
            pl.BlockSpec((BM, K), lambda i: (i, 0)),
            pl.BlockSpec((1, K), lambda i: (0, 0)),
            pl.BlockSpec((1, K), lambda i: (0, 0)),
            pl.BlockSpec((K, N), lambda i: (0, 0)),
            pl.BlockSpec((1, N), lambda i: (0, 0)),
        ],
        out_specs=pl.BlockSpec((BM, N), lambda i: (i, 0)),
        out_shape=jax.ShapeDtypeStruct((M, N), jnp.float32),
    )(x, ln_g.reshape(1, K), ln_b.reshape(1, K), w, b.reshape(1, N))


def _bucket_body(qk_ref, r_ref, o_ref):
    qh = qk_ref[0]
    outs = []
    for n in range(NHASH):
        rot = jnp.dot(qh, r_ref[n], preferred_element_type=jnp.float32)
        both = jnp.concatenate([rot, -rot], axis=-1)
        outs.append(jnp.argmax(both, axis=-1).astype(jnp.int32))
    o_ref[0] = jnp.stack(outs, axis=0)


def _buckets(qkh, R):
    Hh, S, _ = qkh.shape
    return pl.pallas_call(
        _bucket_body,
        grid=(Hh,),
        in_specs=[
            pl.BlockSpec((1, S, DH), lambda h: (h, 0, 0)),
            pl.BlockSpec((NHASH, DH, NB2), lambda h: (0, 0, 0)),
        ],
        out_specs=pl.BlockSpec((1, NHASH, S), lambda h: (h, 0, 0)),
        out_shape=jax.ShapeDtypeStruct((Hh, NHASH, S), jnp.int32),
    )(qkh, R)


def _lsh_body(q_ref, v_ref, o_ref, *, S):
    q = q_ref[0]
    v = v_ref[0]
    n = jnp.sqrt(jnp.sum(q * q, axis=-1, keepdims=True))
    k = q / (n + 1e-6)
    B = BUCKET
    ng = S // (2 * B)
    cq = jax.lax.broadcasted_iota(jnp.int32, (2 * B, 3 * B), 0) // B
    ck = jax.lax.broadcasted_iota(jnp.int32, (2 * B, 3 * B), 1) // B
    allowed = (ck - cq >= 0) & (ck - cq <= 1)
    for i in range(ng):
        qg = q[i * 2 * B:(i + 1) * 2 * B]
        if i == 0:
            kw = jnp.concatenate([k[S - B:], k[:2 * B]], axis=0)
            vw = jnp.concatenate([v[S - B:], v[:2 * B]], axis=0)
        else:
            lo = i * 2 * B - B
            kw = k[lo:lo + 3 * B]
            vw = v[lo:lo + 3 * B]
        sc = jax.lax.dot_general(
            qg.astype(jnp.bfloat16), kw.astype(jnp.bfloat16),
            (((1,), (1,)), ((), ())),
            preferred_element_type=jnp.float32) / SCALE
        sc = jnp.where(allowed, sc, -1e9)
        m = jnp.max(sc, axis=-1, keepdims=True)
        e = jnp.exp(sc - m)
        a = e / jnp.sum(e, axis=-1, keepdims=True)
        o_ref[0, i * 2 * B:(i + 1) * 2 * B, :] = jnp.dot(
            a.astype(jnp.bfloat16), vw.astype(jnp.bfloat16),
            preferred_element_type=jnp.float32)


def _lsh_attn_sorted(sq, sv):
    G, S, _ = sq.shape
    return pl.pallas_call(
        functools.partial(_lsh_body, S=S),
        grid=(G,),
        in_specs=[
            pl.BlockSpec((1, S, DH), lambda g: (g, 0, 0)),
            pl.BlockSpec((1, S, DH), lambda g: (g, 0, 0)),
        ],
        out_specs=pl.BlockSpec((1, S, DH), lambda g: (g, 0, 0)),
        out_shape=jax.ShapeDtypeStruct((G, S, DH), jnp.float32),
    )(sq, sv)


def _roll_rows(a, t):
    t = t % a.shape[0]
    if t == 0:
        return a
    return jnp.concatenate([a[-t:], a[:-t]], axis=0)


def _local_body(q_ref, v_ref, o_ref, *, S):
    q = q_ref[0]
    v = v_ref[0]
    n = jnp.sqrt(jnp.sum(q * q, axis=-1, keepdims=True))
    k = q / (n + 1e-6)
    pos = jax.lax.broadcasted_iota(jnp.int32, (S, 1), 0)
    shifts = list(range(-RADIUS, RADIUS + 1))
    scs = []
    for s in shifts:
        kw = _roll_rows(k, -s)
        sc = jnp.sum(q * kw, axis=-1, keepdims=True) / SCALE
        valid = (pos + s >= 0) & (pos + s < S)
        scs.append(jnp.where(valid, sc, -1e9))
    m = scs[0]
    for sc in scs[1:]:
        m = jnp.maximum(m, sc)
    es = [jnp.exp(sc - m) for sc in scs]
    denom = es[0]
    for e in es[1:]:
        denom = denom + e
    acc = jnp.zeros_like(v)
    for s, e in zip(shifts, es):
        acc = acc + (e / denom) * _roll_rows(v, -s)
    o_ref[0] = acc


def _local_attn(qkh, vh):
    Hh, S, _ = qkh.shape
    return pl.pallas_call(
        functools.partial(_local_body, S=S),
        grid=(Hh,),
        in_specs=[
            pl.BlockSpec((1, S, DH), lambda h: (h, 0, 0)),
            pl.BlockSpec((1, S, DH), lambda h: (h, 0, 0)),
        ],
        out_specs=pl.BlockSpec((1, S, DH), lambda h: (h, 0, 0)),
        out_shape=jax.ShapeDtypeStruct((Hh, S, DH), jnp.float32),
    )(qkh, vh)


def _heads(x2d, W, b0, S, bf16=False):
    y = _dense(x2d, W, b0, bf16=bf16)
    return y.reshape(S, H, DH).transpose(1, 0, 2)


def _attn_layer(x2d, lp, S, zeros_d):
    qkh = _heads(x2d, lp['Wqk'], zeros_d, S)
    vh = _heads(x2d, lp['Wv'], zeros_d, S, bf16=True)

    bks = _buckets(qkh, lp['R'])
    bks = bks.transpose(1, 0, 2)
    ticker = jnp.arange(S, dtype=jnp.int32)
    perm = jnp.argsort(bks * S + ticker[None, None, :], axis=-1)
    inv = jnp.argsort(perm, axis=-1)

    sq = jnp.take_along_axis(
        jnp.broadcast_to(qkh[None], (NHASH, H, S, DH)), perm[..., None], axis=2)
    sv = jnp.take_along_axis(
        jnp.broadcast_to(vh[None], (NHASH, H, S, DH)), perm[..., None], axis=2)

    so = _lsh_attn_sorted(sq.reshape(NHASH * H, S, DH),
                          sv.reshape(NHASH * H, S, DH))
    so = so.reshape(NHASH, H, S, DH)
    o = jnp.take_along_axis(so, inv[..., None], axis=2)
    lsh = jnp.mean(o, axis=0)

    loc = _local_attn(qkh, vh)

    g = jax.nn.sigmoid(lp['router'][:, :S])
    comb = g[:, :, None] * lsh + (1.0 - g)[:, :, None] * loc
    o2d = comb.transpose(1, 0, 2).reshape(S, H * DH)
    out = _dense(o2d, lp['Wo'], zeros_d, bf16=True)
    reg = jnp.mean(g * (1.0 - g))
    return out, reg


def _run(src, tgt, params):
    S = src.shape[1]
    D = params['emb'].shape[1]
    x = params['emb'][src[0]] + params['pos'][0, :S, :]
    zeros_d = jnp.zeros((D,), jnp.float32)

    x1 = x
    x2 = jnp.zeros_like(x)
    total_reg = jnp.zeros((), jnp.float32)
    for lp in params['layers']:
        a, reg = _attn_layer(x2, lp, S, zeros_d)
        y1 = x1 + jax.nn.sigmoid(lp['gf'])[None, :] * a
        h = _ln_dense_relu(y1, lp['ln_g'], lp['ln_b'], lp['W1'], lp['b1'])
        ffn = _dense(h, lp['W2'], lp['b2'], bf16=True)
        y2 = x2 + jax.nn.sigmoid(lp['gg'])[None, :] * ffn
        x1, x2 = y1, y2
        total_reg = total_reg + reg

    out = (x1 + x2) / 2.0
    out = out[: tgt.shape[1], :]
    logits = _dense(out, params['Wout'], params['bout'], bn=1280, bf16=True)
    return logits[None], total_reg


@jax.jit
def kernel(src, tgt, params):
    return _run(src, tgt, params)


# speedup vs baseline: 1.3833x; 1.1113x over previous
import functools
import jax
import jax.numpy as jnp
from jax.experimental import pallas as pl

H = 16
DH = 64
BUCKET = 64
NHASH = 4
RADIUS = 4
NB2 = 16
SCALE = 8.0
_LSH_GRP = 4

BM = 256


def _dense_body(x_ref, w_ref, b_ref, o_ref, *, relu, bf16):
    x = x_ref[...]
    w = w_ref[...]
    if bf16:
        x = x.astype(jnp.bfloat16)
        w = w.astype(jnp.bfloat16)
    acc = jnp.dot(x, w, preferred_element_type=jnp.float32)
    acc = acc + b_ref[0][None, :]
    if relu:
        acc = jnp.maximum(acc, 0.0)
    o_ref[...] = acc


def _dense(x, w, b, relu=False, bn=None, bf16=False):
    M, K = x.shape
    K2, N = w.shape
    assert K == K2
    if bn is None:
        bn = N
    grid = (M // BM, N // bn)
    return pl.pallas_call(
        functools.partial(_dense_body, relu=relu, bf16=bf16),
        grid=grid,
        in_specs=[
            pl.BlockSpec((BM, K), lambda i, j: (i, 0)),
            pl.BlockSpec((K, bn), lambda i, j: (0, j)),
            pl.BlockSpec((1, bn), lambda i, j: (0, j)),
        ],
        out_specs=pl.BlockSpec((BM, bn), lambda i, j: (i, j)),
        out_shape=jax.ShapeDtypeStruct((M, N), jnp.float32),
    )(x, w, b.reshape(1, N))


def _ln_dense_body(x_ref, g_ref, bt_ref, w_ref, b_ref, o_ref):
    xb = x_ref[...]
    mu = jnp.mean(xb, axis=-1, keepdims=True)
    xc = xb - mu
    var = jnp.mean(xc * xc, axis=-1, keepdims=True)
    xn = xc * jax.lax.rsqrt(var + 1e-5) * g_ref[0][None, :] + bt_ref[0][None, :]
    acc = jnp.dot(xn.astype(jnp.bfloat16), w_ref[...].astype(jnp.bfloat16),
                  preferred_element_type=jnp.float32)
    acc = acc + b_ref[0][None, :]
    o_ref[...] = jnp.maximum(acc, 0.0)


def _ln_dense_relu(x, ln_g, ln_b, w, b):
    M, K = x.shape
    _, N = w.shape
    grid = (M // BM,)
    return pl.pallas_call(
        _ln_dense_body,
        grid=grid,
        in_specs=[
            pl.BlockSpec((BM, K), lambda i: (i, 0)),
            pl.BlockSpec((1, K), lambda i: (0, 0)),
            pl.BlockSpec((1, K), lambda i: (0, 0)),
            pl.BlockSpec((K, N), lambda i: (0, 0)),
            pl.BlockSpec((1, N), lambda i: (0, 0)),
        ],
        out_specs=pl.BlockSpec((BM, N), lambda i: (i, 0)),
        out_shape=jax.ShapeDtypeStruct((M, N), jnp.float32),
    )(x, ln_g.reshape(1, K), ln_b.reshape(1, K), w, b.reshape(1, N))


def _bucket_body(qk_ref, r_ref, o_ref):
    qh = qk_ref[0]
    outs = []
    for n in range(NHASH):
        rot = jnp.dot(qh, r_ref[n], preferred_element_type=jnp.float32)
        both = jnp.concatenate([rot, -rot], axis=-1)
        outs.append(jnp.argmax(both, axis=-1).astype(jnp.int32))
    o_ref[0] = jnp.stack(outs, axis=0)


def _buckets(qkh, R):
    Hh, S, _ = qkh.shape
    return pl.pallas_call(
        _bucket_body,
        grid=(Hh,),
        in_specs=[
            pl.BlockSpec((1, S, DH), lambda h: (h, 0, 0)),
            pl.BlockSpec((NHASH, DH, NB2), lambda h: (0, 0, 0)),
        ],
        out_specs=pl.BlockSpec((1, NHASH, S), lambda h: (h, 0, 0)),
        out_shape=jax.ShapeDtypeStruct((Hh, NHASH, S), jnp.int32),
    )(qkh, R)


def _lsh_body(q_ref, v_ref, o_ref, *, S):
    q = q_ref[0]
    v = v_ref[0]
    n = jnp.sqrt(jnp.sum(q * q, axis=-1, keepdims=True))
    k = q / (n + 1e-6)
    B = BUCKET
    G = _LSH_GRP
    ng = S // (G * B)
    cq = jax.lax.broadcasted_iota(jnp.int32, (G * B, (G + 1) * B), 0) // B
    ck = jax.lax.broadcasted_iota(jnp.int32, (G * B, (G + 1) * B), 1) // B
    allowed = (ck - cq >= 0) & (ck - cq <= 1)
    for i in range(ng):
        qg = q[i * G * B:(i + 1) * G * B]
        if i == 0:
            kw = jnp.concatenate([k[S - B:], k[:G * B]], axis=0)
            vw = jnp.concatenate([v[S - B:], v[:G * B]], axis=0)
        else:
            lo = i * G * B - B
            kw = k[lo:lo + (G + 1) * B]
            vw = v[lo:lo + (G + 1) * B]
        sc = jax.lax.dot_general(
            qg.astype(jnp.bfloat16), kw.astype(jnp.bfloat16),
            (((1,), (1,)), ((), ())),
            preferred_element_type=jnp.float32) / SCALE
        sc = jnp.where(allowed, sc, -1e9)
        m = jnp.max(sc, axis=-1, keepdims=True)
        e = jnp.exp(sc - m)
        a = e / jnp.sum(e, axis=-1, keepdims=True)
        o_ref[0, i * G * B:(i + 1) * G * B, :] = jnp.dot(
            a.astype(jnp.bfloat16), vw.astype(jnp.bfloat16),
            preferred_element_type=jnp.float32)


def _lsh_attn_sorted(sq, sv):
    G, S, _ = sq.shape
    return pl.pallas_call(
        functools.partial(_lsh_body, S=S),
        grid=(G,),
        in_specs=[
            pl.BlockSpec((1, S, DH), lambda g: (g, 0, 0)),
            pl.BlockSpec((1, S, DH), lambda g: (g, 0, 0)),
        ],
        out_specs=pl.BlockSpec((1, S, DH), lambda g: (g, 0, 0)),
        out_shape=jax.ShapeDtypeStruct((G, S, DH), jnp.float32),
    )(sq, sv)


def _roll_rows(a, t):
    t = t % a.shape[0]
    if t == 0:
        return a
    return jnp.concatenate([a[-t:], a[:-t]], axis=0)


def _local_body(q_ref, v_ref, o_ref, *, S):
    q = q_ref[0]
    v = v_ref[0]
    n = jnp.sqrt(jnp.sum(q * q, axis=-1, keepdims=True))
    k = q / (n + 1e-6)
    pos = jax.lax.broadcasted_iota(jnp.int32, (S, 1), 0)
    shifts = list(range(-RADIUS, RADIUS + 1))
    scs = []
    for s in shifts:
        kw = _roll_rows(k, -s)
        sc = jnp.sum(q * kw, axis=-1, keepdims=True) / SCALE
        valid = (pos + s >= 0) & (pos + s < S)
        scs.append(jnp.where(valid, sc, -1e9))
    m = scs[0]
    for sc in scs[1:]:
        m = jnp.maximum(m, sc)
    es = [jnp.exp(sc - m) for sc in scs]
    denom = es[0]
    for e in es[1:]:
        denom = denom + e
    acc = jnp.zeros_like(v)
    for s, e in zip(shifts, es):
        acc = acc + (e / denom) * _roll_rows(v, -s)
    o_ref[0] = acc


def _local_attn(qkh, vh):
    Hh, S, _ = qkh.shape
    return pl.pallas_call(
        functools.partial(_local_body, S=S),
        grid=(Hh,),
        in_specs=[
            pl.BlockSpec((1, S, DH), lambda h: (h, 0, 0)),
            pl.BlockSpec((1, S, DH), lambda h: (h, 0, 0)),
        ],
        out_specs=pl.BlockSpec((1, S, DH), lambda h: (h, 0, 0)),
        out_shape=jax.ShapeDtypeStruct((Hh, S, DH), jnp.float32),
    )(qkh, vh)


def _heads(x2d, W, b0, S, bf16=False):
    y = _dense(x2d, W, b0, bf16=bf16)
    return y.reshape(S, H, DH).transpose(1, 0, 2)


def _attn_layer(x2d, lp, S, zeros_d):
    qkh = _heads(x2d, lp['Wqk'], zeros_d, S)
    vh = _heads(x2d, lp['Wv'], zeros_d, S, bf16=True)

    bks = _buckets(qkh, lp['R'])
    bks = bks.transpose(1, 0, 2)
    ticker = jnp.arange(S, dtype=jnp.int32)
    perm = jnp.argsort(bks * S + ticker[None, None, :], axis=-1)
    inv = jnp.argsort(perm, axis=-1)

    sq = jnp.take_along_axis(
        jnp.broadcast_to(qkh[None], (NHASH, H, S, DH)), perm[..., None], axis=2)
    sv = jnp.take_along_axis(
        jnp.broadcast_to(vh[None], (NHASH, H, S, DH)), perm[..., None], axis=2)

    so = _lsh_attn_sorted(sq.reshape(NHASH * H, S, DH),
                          sv.reshape(NHASH * H, S, DH))
    so = so.reshape(NHASH, H, S, DH)
    o = jnp.take_along_axis(so, inv[..., None], axis=2)
    lsh = jnp.mean(o, axis=0)

    loc = _local_attn(qkh, vh)

    g = jax.nn.sigmoid(lp['router'][:, :S])
    comb = g[:, :, None] * lsh + (1.0 - g)[:, :, None] * loc
    o2d = comb.transpose(1, 0, 2).reshape(S, H * DH)
    out = _dense(o2d, lp['Wo'], zeros_d, bf16=True)
    reg = jnp.mean(g * (1.0 - g))
    return out, reg


def _run(src, tgt, params):
    S = src.shape[1]
    D = params['emb'].shape[1]
    x = params['emb'][src[0]] + params['pos'][0, :S, :]
    zeros_d = jnp.zeros((D,), jnp.float32)

    x1 = x
    x2 = jnp.zeros_like(x)
    total_reg = jnp.zeros((), jnp.float32)
    for lp in params['layers']:
        a, reg = _attn_layer(x2, lp, S, zeros_d)
        y1 = x1 + jax.nn.sigmoid(lp['gf'])[None, :] * a
        h = _ln_dense_relu(y1, lp['ln_g'], lp['ln_b'], lp['W1'], lp['b1'])
        ffn = _dense(h, lp['W2'], lp['b2'], bf16=True)
        y2 = x2 + jax.nn.sigmoid(lp['gg'])[None, :] * ffn
        x1, x2 = y1, y2
        total_reg = total_reg + reg

    out = (x1 + x2) / 2.0
    out = out[: tgt.shape[1], :]
    logits = _dense(out, params['Wout'], params['bout'], bn=1280, bf16=True)
    return logits[None], total_reg


@jax.jit
def kernel(src, tgt, params):
    return _run(src, tgt, params)


# speedup vs baseline: 1.3954x; 1.0087x over previous
import functools
import jax
import jax.numpy as jnp
from jax.experimental import pallas as pl

H = 16
DH = 64
BUCKET = 64
NHASH = 4
RADIUS = 4
NB2 = 16
SCALE = 8.0
_LSH_GRP = 8

BM = 256


def _dense_body(x_ref, w_ref, b_ref, o_ref, *, relu, bf16):
    x = x_ref[...]
    w = w_ref[...]
    if bf16:
        x = x.astype(jnp.bfloat16)
        w = w.astype(jnp.bfloat16)
    acc = jnp.dot(x, w, preferred_element_type=jnp.float32)
    acc = acc + b_ref[0][None, :]
    if relu:
        acc = jnp.maximum(acc, 0.0)
    o_ref[...] = acc


def _dense(x, w, b, relu=False, bn=None, bf16=False):
    M, K = x.shape
    K2, N = w.shape
    assert K == K2
    if bn is None:
        bn = N
    grid = (M // BM, N // bn)
    return pl.pallas_call(
        functools.partial(_dense_body, relu=relu, bf16=bf16),
        grid=grid,
        in_specs=[
            pl.BlockSpec((BM, K), lambda i, j: (i, 0)),
            pl.BlockSpec((K, bn), lambda i, j: (0, j)),
            pl.BlockSpec((1, bn), lambda i, j: (0, j)),
        ],
        out_specs=pl.BlockSpec((BM, bn), lambda i, j: (i, j)),
        out_shape=jax.ShapeDtypeStruct((M, N), jnp.float32),
    )(x, w, b.reshape(1, N))


def _ln_dense_body(x_ref, g_ref, bt_ref, w_ref, b_ref, o_ref):
    xb = x_ref[...]
    mu = jnp.mean(xb, axis=-1, keepdims=True)
    xc = xb - mu
    var = jnp.mean(xc * xc, axis=-1, keepdims=True)
    xn = xc * jax.lax.rsqrt(var + 1e-5) * g_ref[0][None, :] + bt_ref[0][None, :]
    acc = jnp.dot(xn.astype(jnp.bfloat16), w_ref[...].astype(jnp.bfloat16),
                  preferred_element_type=jnp.float32)
    acc = acc + b_ref[0][None, :]
    o_ref[...] = jnp.maximum(acc, 0.0)


def _ln_dense_relu(x, ln_g, ln_b, w, b):
    M, K = x.shape
    _, N = w.shape
    grid = (M // BM,)
    return pl.pallas_call(
        _ln_dense_body,
        grid=grid,
        in_specs=[
            pl.BlockSpec((BM, K), lambda i: (i, 0)),
            pl.BlockSpec((1, K), lambda i: (0, 0)),
            pl.BlockSpec((1, K), lambda i: (0, 0)),
            pl.BlockSpec((K, N), lambda i: (0, 0)),
            pl.BlockSpec((1, N), lambda i: (0, 0)),
        ],
        out_specs=pl.BlockSpec((BM, N), lambda i: (i, 0)),
        out_shape=jax.ShapeDtypeStruct((M, N), jnp.float32),
    )(x, ln_g.reshape(1, K), ln_b.reshape(1, K), w, b.reshape(1, N))


def _bucket_body(qk_ref, r_ref, o_ref):
    qh = qk_ref[0]
    outs = []
    for n in range(NHASH):
        rot = jnp.dot(qh, r_ref[n], preferred_element_type=jnp.float32)
        both = jnp.concatenate([rot, -rot], axis=-1)
        outs.append(jnp.argmax(both, axis=-1).astype(jnp.int32))
    o_ref[0] = jnp.stack(outs, axis=0)


def _buckets(qkh, R):
    Hh, S, _ = qkh.shape
    return pl.pallas_call(
        _bucket_body,
        grid=(Hh,),
        in_specs=[
            pl.BlockSpec((1, S, DH), lambda h: (h, 0, 0)),
            pl.BlockSpec((NHASH, DH, NB2), lambda h: (0, 0, 0)),
        ],
        out_specs=pl.BlockSpec((1, NHASH, S), lambda h: (h, 0, 0)),
        out_shape=jax.ShapeDtypeStruct((Hh, NHASH, S), jnp.int32),
    )(qkh, R)


def _lsh_body(q_ref, v_ref, o_ref, *, S):
    q = q_ref[0]
    v = v_ref[0]
    n = jnp.sqrt(jnp.sum(q * q, axis=-1, keepdims=True))
    k = q / (n + 1e-6)
    B = BUCKET
    G = _LSH_GRP
    ng = S // (G * B)
    cq = jax.lax.broadcasted_iota(jnp.int32, (G * B, (G + 1) * B), 0) // B
    ck = jax.lax.broadcasted_iota(jnp.int32, (G * B, (G + 1) * B), 1) // B
    allowed = (ck - cq >= 0) & (ck - cq <= 1)
    for i in range(ng):
        qg = q[i * G * B:(i + 1) * G * B]
        if i == 0:
            kw = jnp.concatenate([k[S - B:], k[:G * B]], axis=0)
            vw = jnp.concatenate([v[S - B:], v[:G * B]], axis=0)
        else:
            lo = i * G * B - B
            kw = k[lo:lo + (G + 1) * B]
            vw = v[lo:lo + (G + 1) * B]
        sc = jax.lax.dot_general(
            qg.astype(jnp.bfloat16), kw.astype(jnp.bfloat16),
            (((1,), (1,)), ((), ())),
            preferred_element_type=jnp.float32) / SCALE
        sc = jnp.where(allowed, sc, -1e9)
        m = jnp.max(sc, axis=-1, keepdims=True)
        e = jnp.exp(sc - m)
        a = e / jnp.sum(e, axis=-1, keepdims=True)
        o_ref[0, i * G * B:(i + 1) * G * B, :] = jnp.dot(
            a.astype(jnp.bfloat16), vw.astype(jnp.bfloat16),
            preferred_element_type=jnp.float32)


def _lsh_attn_sorted(sq, sv):
    G, S, _ = sq.shape
    return pl.pallas_call(
        functools.partial(_lsh_body, S=S),
        grid=(G,),
        in_specs=[
            pl.BlockSpec((1, S, DH), lambda g: (g, 0, 0)),
            pl.BlockSpec((1, S, DH), lambda g: (g, 0, 0)),
        ],
        out_specs=pl.BlockSpec((1, S, DH), lambda g: (g, 0, 0)),
        out_shape=jax.ShapeDtypeStruct((G, S, DH), jnp.float32),
    )(sq, sv)


def _roll_rows(a, t):
    t = t % a.shape[0]
    if t == 0:
        return a
    return jnp.concatenate([a[-t:], a[:-t]], axis=0)


def _local_body(q_ref, v_ref, o_ref, *, S):
    q = q_ref[0]
    v = v_ref[0]
    n = jnp.sqrt(jnp.sum(q * q, axis=-1, keepdims=True))
    k = q / (n + 1e-6)
    pos = jax.lax.broadcasted_iota(jnp.int32, (S, 1), 0)
    shifts = list(range(-RADIUS, RADIUS + 1))
    scs = []
    for s in shifts:
        kw = _roll_rows(k, -s)
        sc = jnp.sum(q * kw, axis=-1, keepdims=True) / SCALE
        valid = (pos + s >= 0) & (pos + s < S)
        scs.append(jnp.where(valid, sc, -1e9))
    m = scs[0]
    for sc in scs[1:]:
        m = jnp.maximum(m, sc)
    es = [jnp.exp(sc - m) for sc in scs]
    denom = es[0]
    for e in es[1:]:
        denom = denom + e
    acc = jnp.zeros_like(v)
    for s, e in zip(shifts, es):
        acc = acc + (e / denom) * _roll_rows(v, -s)
    o_ref[0] = acc


def _local_attn(qkh, vh):
    Hh, S, _ = qkh.shape
    return pl.pallas_call(
        functools.partial(_local_body, S=S),
        grid=(Hh,),
        in_specs=[
            pl.BlockSpec((1, S, DH), lambda h: (h, 0, 0)),
            pl.BlockSpec((1, S, DH), lambda h: (h, 0, 0)),
        ],
        out_specs=pl.BlockSpec((1, S, DH), lambda h: (h, 0, 0)),
        out_shape=jax.ShapeDtypeStruct((Hh, S, DH), jnp.float32),
    )(qkh, vh)


def _heads(x2d, W, b0, S, bf16=False):
    y = _dense(x2d, W, b0, bf16=bf16)
    return y.reshape(S, H, DH).transpose(1, 0, 2)


def _attn_layer(x2d, lp, S, zeros_d):
    qkh = _heads(x2d, lp['Wqk'], zeros_d, S)
    vh = _heads(x2d, lp['Wv'], zeros_d, S, bf16=True)

    bks = _buckets(qkh, lp['R'])
    bks = bks.transpose(1, 0, 2)
    ticker = jnp.arange(S, dtype=jnp.int32)
    perm = jnp.argsort(bks * S + ticker[None, None, :], axis=-1)
    inv = jnp.argsort(perm, axis=-1)

    sq = jnp.take_along_axis(
        jnp.broadcast_to(qkh[None], (NHASH, H, S, DH)), perm[..., None], axis=2)
    sv = jnp.take_along_axis(
        jnp.broadcast_to(vh[None], (NHASH, H, S, DH)), perm[..., None], axis=2)

    so = _lsh_attn_sorted(sq.reshape(NHASH * H, S, DH),
                          sv.reshape(NHASH * H, S, DH))
    so = so.reshape(NHASH, H, S, DH)
    o = jnp.take_along_axis(so, inv[..., None], axis=2)
    lsh = jnp.mean(o, axis=0)

    loc = _local_attn(qkh, vh)

    g = jax.nn.sigmoid(lp['router'][:, :S])
    comb = g[:, :, None] * lsh + (1.0 - g)[:, :, None] * loc
    o2d = comb.transpose(1, 0, 2).reshape(S, H * DH)
    out = _dense(o2d, lp['Wo'], zeros_d, bf16=True)
    reg = jnp.mean(g * (1.0 - g))
    return out, reg


def _run(src, tgt, params):
    S = src.shape[1]
    D = params['emb'].shape[1]
    x = params['emb'][src[0]] + params['pos'][0, :S, :]
    zeros_d = jnp.zeros((D,), jnp.float32)

    x1 = x
    x2 = jnp.zeros_like(x)
    total_reg = jnp.zeros((), jnp.float32)
    for lp in params['layers']:
        a, reg = _attn_layer(x2, lp, S, zeros_d)
        y1 = x1 + jax.nn.sigmoid(lp['gf'])[None, :] * a
        h = _ln_dense_relu(y1, lp['ln_g'], lp['ln_b'], lp['W1'], lp['b1'])
        ffn = _dense(h, lp['W2'], lp['b2'], bf16=True)
        y2 = x2 + jax.nn.sigmoid(lp['gg'])[None, :] * ffn
        x1, x2 = y1, y2
        total_reg = total_reg + reg

    out = (x1 + x2) / 2.0
    out = out[: tgt.shape[1], :]
    logits = _dense(out, params['Wout'], params['bout'], bn=1280, bf16=True)
    return logits[None], total_reg


@jax.jit
def kernel(src, tgt, params):
    return _run(src, tgt, params)
